# Initial kernel scaffold; baseline (speedup 1.0000x reference)
import jax
import jax.numpy as jnp
from jax.experimental import pallas as pl


def kernel(x, edge_index, W_self, b_self, W_neigh, b_neigh, W_var, b_var):
    raise NotImplementedError("write your pallas kernel here")


# speedup vs baseline: 5.0489x; 5.0489x over previous
import functools

import jax
import jax.numpy as jnp
from jax import lax
from jax.experimental import pallas as pl
from jax.experimental.pallas import tpu as pltpu
from jax.experimental.pallas import tpu_sc as plsc

NC = 2
NS = 16
L = 16

_SCALE = float(128) ** (-0.5)


def _zero_fill(buf, n_vecs):
    zeros = jnp.zeros((L,), jnp.float32)

    def body(k, _):
        buf[pl.ds(k * L, L)] = zeros
        return 0

    lax.fori_loop(0, n_vecs, body, 0)


def _zero_fill2d(buf, nrows, rowlen):
    zeros = jnp.zeros((L,), jnp.float32)

    def body(r, _):
        for j in range(rowlen // L):
            buf[r, pl.ds(j * L, L)] = zeros
        return 0

    lax.fori_loop(0, nrows, body, 0)


def _edge_kernel(np_pad, e_per_w, nchunks, b,
                 x_hbm, src_hbm, dst_hbm,
                 eatt_hbm, sum_hbm, den_hbm, cnt_hbm,
                 xs_v, xd_v, src_v, dst_v, ebuf, ones_v, zbuf, zflat,
                 sum_sh, den_sh, cnt_sh, sem1, sem2):
    c = lax.axis_index("c")
    s = lax.axis_index("s")
    w = c * NS + s
    rows_per_s = np_pad // NS

    _zero_fill2d(zbuf, b, 128)
    _zero_fill(zflat, rows_per_s // L)
    for k in range(5):
        ones_v[pl.ds(k * L, L)] = jnp.ones((L,), jnp.float32)
    n_zcopy = rows_per_s // b
    for t in range(n_zcopy):
        pltpu.sync_copy(zbuf, sum_sh.at[pl.ds(s * rows_per_s + t * b, b)])
    pltpu.sync_copy(zflat, den_sh.at[pl.ds(s * rows_per_s, rows_per_s)])
    pltpu.sync_copy(zflat, cnt_sh.at[pl.ds(s * rows_per_s, rows_per_s)])
    plsc.subcore_barrier()

    m15 = lax.broadcasted_iota(jnp.int32, (L,), 0) == 15

    def chunk(i, _):
        base = w * e_per_w + i * b
        pltpu.sync_copy(src_hbm.at[pl.ds(base, b)], src_v)
        pltpu.sync_copy(dst_hbm.at[pl.ds(base, b)], dst_v)
        d1 = pltpu.async_copy(x_hbm.at[src_v], xs_v, sem1)
        d2 = pltpu.async_copy(x_hbm.at[dst_v], xd_v, sem2)
        d1.wait()
        d2.wait()

        def edge(e, _):
            acc = xs_v[e, pl.ds(0, L)] * xd_v[e, pl.ds(0, L)]
            for j in range(1, 8):
                acc = acc + xs_v[e, pl.ds(j * L, L)] * xd_v[e, pl.ds(j * L, L)]
            sc = plsc.cumsum(acc)
            ev = jnp.exp(sc * _SCALE)
            plsc.store_scatter(ebuf, [jnp.full((L,), e, jnp.int32)], ev,
                               mask=m15)
            return 0

        lax.fori_loop(0, b, edge, 0)

        pltpu.sync_copy(ebuf, eatt_hbm.at[pl.ds(base, b)])
        pltpu.sync_copy(ebuf, den_sh.at[dst_v], add=True)
        pltpu.sync_copy(ones_v, cnt_sh.at[dst_v], add=True)
        pltpu.sync_copy(xs_v, sum_sh.at[dst_v], add=True)
        return 0

    lax.fori_loop(0, nchunks, chunk, 0)
    plsc.subcore_barrier()

    r0 = s * rows_per_s
    pltpu.sync_copy(sum_sh.at[pl.ds(r0, rows_per_s)],
                    sum_hbm.at[c, pl.ds(r0, rows_per_s)])
    pltpu.sync_copy(den_sh.at[pl.ds(r0, rows_per_s)],
                    den_hbm.at[c, pl.ds(r0, rows_per_s)])
    pltpu.sync_copy(cnt_sh.at[pl.ds(r0, rows_per_s)],
                    cnt_hbm.at[c, pl.ds(r0, rows_per_s)])


def _feat_kernel(n, np_pad, e_per_s, nchunks, b,
                 xcat_hbm, src_hbm, dst_hbm, eatt_hbm,
                 msg_hbm, sq_hbm,
                 src_v, gsrc_v, dst_v, ebuf, xs_v, msgst, sqst, zbuf,
                 msg_sh, sq_sh, sem1):
    c = lax.axis_index("c")
    s = lax.axis_index("s")
    rows_per_s = np_pad // NS

    _zero_fill2d(zbuf, b, 64)
    n_zcopy = rows_per_s // b
    for t in range(n_zcopy):
        pltpu.sync_copy(zbuf, msg_sh.at[pl.ds(s * rows_per_s + t * b, b)])
        pltpu.sync_copy(zbuf, sq_sh.at[pl.ds(s * rows_per_s + t * b, b)])
    plsc.subcore_barrier()

    off = c * n

    def chunk(i, _):
        base = s * e_per_s + i * b
        pltpu.sync_copy(src_hbm.at[pl.ds(base, b)], src_v)
        pltpu.sync_copy(dst_hbm.at[pl.ds(base, b)], dst_v)
        pltpu.sync_copy(eatt_hbm.at[pl.ds(base, b)], ebuf)
        for k in range(b // L):
            gsrc_v[pl.ds(k * L, L)] = src_v[pl.ds(k * L, L)] + off
        pltpu.async_copy(xcat_hbm.at[gsrc_v], xs_v, sem1).wait()

        def edge(e, _):
            eb = plsc.load_gather(ebuf, [jnp.full((L,), e, jnp.int32)])
            for j in range(4):
                xsj = xs_v[e, pl.ds(j * L, L)]
                msgst[e, pl.ds(j * L, L)] = eb * xsj
                sqst[e, pl.ds(j * L, L)] = xsj * xsj
            return 0

        lax.fori_loop(0, b, edge, 0)

        pltpu.sync_copy(msgst, msg_sh.at[dst_v], add=True)
        pltpu.sync_copy(sqst, sq_sh.at[dst_v], add=True)
        return 0

    lax.fori_loop(0, nchunks, chunk, 0)
    plsc.subcore_barrier()

    r0 = s * rows_per_s
    pltpu.sync_copy(msg_sh.at[pl.ds(r0, rows_per_s)],
                    msg_hbm.at[c, pl.ds(r0, rows_per_s)])
    pltpu.sync_copy(sq_sh.at[pl.ds(r0, rows_per_s)],
                    sq_hbm.at[c, pl.ds(r0, rows_per_s)])


def _combine_kernel(x_ref, msg_ref, sum_ref, sq_ref, den_ref, cnt_ref,
                    w_ref, bias_ref, o_ref):
    msg = msg_ref[...] / (den_ref[...] + 1e-16)
    inv = 1.0 / jnp.maximum(cnt_ref[...], 1.0)
    mean = sum_ref[...] * inv
    var = sq_ref[...] * inv - mean * mean
    h = jnp.concatenate([x_ref[...], msg, var], axis=1)
    o_ref[...] = (jnp.dot(h, w_ref[...], preferred_element_type=jnp.float32)
                  + bias_ref[0:1, :])


def kernel(x, edge_index, W_self, b_self, W_neigh, b_neigh, W_var, b_var):
    n, d = x.shape
    e = edge_index.shape[1]
    assert d == 128
    np_pad = ((n + NS * 16 - 1) // (NS * 16)) * (NS * 16)
    b = 80
    e_per_w = e // (NC * NS)
    e_per_s = e // NS
    assert e_per_w % b == 0 and e_per_s % b == 0

    src = edge_index[0]
    dst = edge_index[1]
    mesh = plsc.VectorSubcoreMesh(core_axis_name="c", subcore_axis_name="s")

    sc_params = pltpu.CompilerParams(needs_layout_passes=False,
                                     use_tc_tiling_on_sc=False)
    k1 = functools.partial(
        pl.kernel,
        compiler_params=sc_params,
        out_type=(
            jax.ShapeDtypeStruct((e,), jnp.float32),
            jax.ShapeDtypeStruct((NC, np_pad, 128), jnp.float32),
            jax.ShapeDtypeStruct((NC, np_pad), jnp.float32),
            jax.ShapeDtypeStruct((NC, np_pad), jnp.float32),
        ),
        mesh=mesh,
        scratch_types=[
            pltpu.VMEM((b, 128), jnp.float32),
            pltpu.VMEM((b, 128), jnp.float32),
            pltpu.VMEM((b,), jnp.int32),
            pltpu.VMEM((b,), jnp.int32),
            pltpu.VMEM((b,), jnp.float32),
            pltpu.VMEM((b,), jnp.float32),
            pltpu.VMEM((b, 128), jnp.float32),
            pltpu.VMEM((640,), jnp.float32),
            pltpu.VMEM_SHARED((np_pad, 128), jnp.float32),
            pltpu.VMEM_SHARED((np_pad,), jnp.float32),
            pltpu.VMEM_SHARED((np_pad,), jnp.float32),
            pltpu.SemaphoreType.DMA,
            pltpu.SemaphoreType.DMA,
        ],
    )(functools.partial(_edge_kernel, np_pad, e_per_w, e_per_w // b, b))
    eatt, sum_p, den_p, cnt_p = k1(x, src, dst)

    xcat = jnp.concatenate([x[:, :64], x[:, 64:]], axis=0)
    k2 = functools.partial(
        pl.kernel,
        compiler_params=sc_params,
        out_type=(
            jax.ShapeDtypeStruct((NC, np_pad, 64), jnp.float32),
            jax.ShapeDtypeStruct((NC, np_pad, 64), jnp.float32),
        ),
        mesh=mesh,
        scratch_types=[
            pltpu.VMEM((b,), jnp.int32),
            pltpu.VMEM((b,), jnp.int32),
            pltpu.VMEM((b,), jnp.int32),
            pltpu.VMEM((b,), jnp.float32),
            pltpu.VMEM((b, 64), jnp.float32),
            pltpu.VMEM((b, 64), jnp.float32),
            pltpu.VMEM((b, 64), jnp.float32),
            pltpu.VMEM((b, 64), jnp.float32),
            pltpu.VMEM_SHARED((np_pad, 64), jnp.float32),
            pltpu.VMEM_SHARED((np_pad, 64), jnp.float32),
            pltpu.SemaphoreType.DMA,
        ],
    )(functools.partial(_feat_kernel, n, np_pad, e_per_s, e_per_s // b, b))
    msg_p, sq_p = k2(xcat, src, dst, eatt)

    sum_src = (sum_p[0] + sum_p[1])[:n]
    den = (den_p[0] + den_p[1])[:n]
    cnt = (cnt_p[0] + cnt_p[1])[:n]
    msg_raw = jnp.concatenate([msg_p[0], msg_p[1]], axis=1)[:n]
    sumsq = jnp.concatenate([sq_p[0], sq_p[1]], axis=1)[:n]
    den_b = jnp.broadcast_to(den[:, None], (n, d))
    cnt_b = jnp.broadcast_to(cnt[:, None], (n, d))
    wcat = jnp.concatenate([W_self.T, W_neigh.T, W_var.T], axis=0)
    bias = jnp.broadcast_to((b_self + b_neigh + b_var)[None, :], (8, d))

    bn = 1000
    grid = n // bn
    out = pl.pallas_call(
        _combine_kernel,
        out_shape=jax.ShapeDtypeStruct((n, d), jnp.float32),
        grid=(grid,),
        in_specs=[
            pl.BlockSpec((bn, d), lambda i: (i, 0)),
            pl.BlockSpec((bn, d), lambda i: (i, 0)),
            pl.BlockSpec((bn, d), lambda i: (i, 0)),
            pl.BlockSpec((bn, d), lambda i: (i, 0)),
            pl.BlockSpec((bn, d), lambda i: (i, 0)),
            pl.BlockSpec((bn, d), lambda i: (i, 0)),
            pl.BlockSpec((3 * d, d), lambda i: (0, 0)),
            pl.BlockSpec((8, d), lambda i: (0, 0)),
        ],
        out_specs=pl.BlockSpec((bn, d), lambda i: (i, 0)),
    )(x, msg_raw, sum_src, sumsq, den_b, cnt_b, wcat, bias)
    return out


# speedup vs baseline: 8.6377x; 1.7108x over previous
import functools

import jax
import jax.numpy as jnp
from jax import lax
from jax.experimental import pallas as pl
from jax.experimental.pallas import tpu as pltpu
from jax.experimental.pallas import tpu_sc as plsc

NC = 2
NS = 16
L = 16

_SCALE = float(128) ** (-0.5)


def _zero_fill(buf, n_vecs):
    zeros = jnp.zeros((L,), jnp.float32)

    def body(k, _):
        buf[pl.ds(k * L, L)] = zeros
        return 0

    lax.fori_loop(0, n_vecs, body, 0)


def _zero_fill2d(buf, nrows, rowlen):
    zeros = jnp.zeros((L,), jnp.float32)

    def body(r, _):
        for j in range(rowlen // L):
            buf[r, pl.ds(j * L, L)] = zeros
        return 0

    lax.fori_loop(0, nrows, body, 0)


def _edge_kernel(np_pad, e_per_w, nchunks, b,
                 x_hbm, src_hbm, dst_hbm,
                 eatt_hbm, sum_hbm, den_hbm, cnt_hbm,
                 xs0, xs1, xs2, xd0, xd1, sv0, sv1, sv2,
                 dv0, dv1, dv2, eb0, eb1, eb2,
                 ones_v, zflat, sum_sh, den_sh, cnt_sh,
                 gs0, gs1, gs2, is0, is1, is2,
                 sl0, sl1, sl2, si0, si1, si2):
    c_ax = lax.axis_index("c")
    s_ax = lax.axis_index("s")
    w = c_ax * NS + s_ax
    rows_per_s = np_pad // NS

    xs = (xs0, xs1, xs2)
    xd = (xd0, xd1)
    sv = (sv0, sv1, sv2)
    dv = (dv0, dv1, dv2)
    eb = (eb0, eb1, eb2)
    gs = (gs0, gs1, gs2)
    isem = (is0, is1, is2)
    slin = (sl0, sl1, sl2)
    sind = (si0, si1, si2)

    _zero_fill2d(xs0, b, 128)
    _zero_fill(zflat, rows_per_s // L)
    for k in range((b + L - 1) // L):
        ones_v[pl.ds(k * L, L)] = jnp.ones((L,), jnp.float32)
    for t in range(rows_per_s // b):
        pltpu.sync_copy(xs0, sum_sh.at[pl.ds(s_ax * rows_per_s + t * b, b)])
    pltpu.sync_copy(zflat, den_sh.at[pl.ds(s_ax * rows_per_s, rows_per_s)])
    pltpu.sync_copy(zflat, cnt_sh.at[pl.ds(s_ax * rows_per_s, rows_per_s)])
    plsc.subcore_barrier()

    m15 = lax.broadcasted_iota(jnp.int32, (L,), 0) == 15

    def base_of(c):
        return w * e_per_w + c * b

    def issue_idx(c, r):
        pltpu.async_copy(src_hbm.at[pl.ds(base_of(c), b)], sv[r], isem[r])
        pltpu.async_copy(dst_hbm.at[pl.ds(base_of(c), b)], dv[r], isem[r])

    def wait_idx(c, r):
        pltpu.make_async_copy(src_hbm.at[pl.ds(base_of(c), b)], sv[r],
                              isem[r]).wait()
        pltpu.make_async_copy(dst_hbm.at[pl.ds(base_of(c), b)], dv[r],
                              isem[r]).wait()

    def issue_gather(r, par):
        pltpu.async_copy(x_hbm.at[sv[r]], xs[r], gs[r])
        pltpu.async_copy(x_hbm.at[dv[r]], xd[par], gs[r])

    def wait_gather(r, par):
        pltpu.make_async_copy(x_hbm.at[sv[r]], xs[r], gs[r]).wait()
        pltpu.make_async_copy(x_hbm.at[dv[r]], xd[par], gs[r]).wait()

    def issue_scat(c, p):
        pltpu.sync_copy(eb[p], eatt_hbm.at[pl.ds(base_of(c), b)])
        pltpu.sync_copy(eb[p], den_sh.at[dv[p]], add=True)
        pltpu.sync_copy(ones_v.at[pl.ds(0, b)], cnt_sh.at[dv[p]], add=True)
        pltpu.sync_copy(xs[p], sum_sh.at[dv[p]], add=True)

    def wait_scat(c, p):
        pass

    def compute(p, par):
        xsp, xdp, ebp = xs[p], xd[par], eb[p]

        def edge(e, _):
            acc = xsp[e, pl.ds(0, L)] * xdp[e, pl.ds(0, L)]
            for j in range(1, 8):
                acc = acc + xsp[e, pl.ds(j * L, L)] * xdp[e, pl.ds(j * L, L)]
            sc = plsc.cumsum(acc)
            ev = jnp.exp(sc * _SCALE)
            plsc.store_scatter(ebp, [jnp.full((L,), e, jnp.int32)], ev,
                               mask=m15)
            return 0

        lax.fori_loop(0, b, edge, 0, unroll=4)

    def body_chunk(c, p, r1, r2, cpar, pre1, pre2, drain):
        wait_gather(p, cpar)
        if pre1:
            wait_idx(c + 1, r1)
            issue_gather(r1, 1 - cpar)
        compute(p, cpar)
        if drain:
            wait_scat(c - 1, r2)
        if pre2:
            issue_idx(c + 2, r2)
        issue_scat(c, p)

    issue_idx(0, 0)
    wait_idx(0, 0)
    issue_gather(0, 0)
    issue_idx(1, 1)

    npeel = 6
    nmain = npeel + ((nchunks - npeel - 2) // 6) * 6
    for c in range(npeel):
        body_chunk(c, c % 3, (c + 1) % 3, (c + 2) % 3, c % 2,
                   True, True, c >= 1)

    def main(k, _):
        for j in range(6):
            c = npeel + k * 6 + j
            jj = npeel + j
            body_chunk(c, jj % 3, (jj + 1) % 3, (jj + 2) % 3, jj % 2,
                       True, True, True)
        return 0

    lax.fori_loop(0, (nmain - npeel) // 6, main, 0)
    for c in range(nmain, nchunks):
        body_chunk(c, c % 3, (c + 1) % 3, (c + 2) % 3, c % 2,
                   c + 1 < nchunks, c + 2 < nchunks, True)
    wait_scat(nchunks - 1, (nchunks - 1) % 3)
    plsc.subcore_barrier()

    r0 = s_ax * rows_per_s
    pltpu.sync_copy(sum_sh.at[pl.ds(r0, rows_per_s)],
                    sum_hbm.at[c_ax, pl.ds(r0, rows_per_s)])
    pltpu.sync_copy(den_sh.at[pl.ds(r0, rows_per_s)],
                    den_hbm.at[c_ax, pl.ds(r0, rows_per_s)])
    pltpu.sync_copy(cnt_sh.at[pl.ds(r0, rows_per_s)],
                    cnt_hbm.at[c_ax, pl.ds(r0, rows_per_s)])


def _feat_kernel(n, np_pad, e_per_s, nchunks, b,
                 xcat_hbm, src_hbm, dst_hbm, eatt_hbm,
                 mq_hbm,
                 sv0, sv1, sv2, gv0, gv1, gv2, dv0, dv1, dv2,
                 eb0, eb1, eb2, xs0, xs1, st0, st1, st2,
                 mq_sh,
                 gs0, gs1, gs2, is0, is1, is2, si0, si1, si2):
    c_ax = lax.axis_index("c")
    s_ax = lax.axis_index("s")
    rows_per_s = np_pad // NS

    sv = (sv0, sv1, sv2)
    gv = (gv0, gv1, gv2)
    dv = (dv0, dv1, dv2)
    eb = (eb0, eb1, eb2)
    xs = (xs0, xs1)
    st = (st0, st1, st2)
    gs = (gs0, gs1, gs2)
    isem = (is0, is1, is2)
    sind = (si0, si1, si2)

    _zero_fill2d(st0, b, 128)
    for t in range(rows_per_s // b):
        pltpu.sync_copy(st0, mq_sh.at[pl.ds(s_ax * rows_per_s + t * b, b)])
    plsc.subcore_barrier()

    off = c_ax * n

    def base_of(c):
        return s_ax * e_per_s + c * b

    def issue_idx(c, r):
        pltpu.async_copy(src_hbm.at[pl.ds(base_of(c), b)], sv[r], isem[r])
        pltpu.async_copy(dst_hbm.at[pl.ds(base_of(c), b)], dv[r], isem[r])
        pltpu.async_copy(eatt_hbm.at[pl.ds(base_of(c), b)], eb[r], isem[r])

    def wait_idx(c, r):
        pltpu.make_async_copy(src_hbm.at[pl.ds(base_of(c), b)], sv[r],
                              isem[r]).wait()
        pltpu.make_async_copy(dst_hbm.at[pl.ds(base_of(c), b)], dv[r],
                              isem[r]).wait()
        pltpu.make_async_copy(eatt_hbm.at[pl.ds(base_of(c), b)], eb[r],
                              isem[r]).wait()

    def issue_gather(r, par):
        for k in range(b // L):
            gv[r][pl.ds(k * L, L)] = sv[r][pl.ds(k * L, L)] + off
        pltpu.async_copy(xcat_hbm.at[gv[r]], xs[par], gs[r])

    def wait_gather(r, par):
        pltpu.make_async_copy(xcat_hbm.at[gv[r]], xs[par], gs[r]).wait()

    def issue_scat(p):
        pltpu.sync_copy(st[p], mq_sh.at[dv[p]], add=True)

    def wait_scat(p):
        pass

    def compute(p, par):
        xsp, ebp, stp = xs[par], eb[p], st[p]

        def edge(e, _):
            ebc = plsc.load_gather(ebp, [jnp.full((L,), e, jnp.int32)])
            for j in range(4):
                xsj = xsp[e, pl.ds(j * L, L)]
                stp[e, pl.ds(j * L, L)] = ebc * xsj
                stp[e, pl.ds(64 + j * L, L)] = xsj * xsj
            return 0

        lax.fori_loop(0, b, edge, 0, unroll=4)

    def body_chunk(c, p, r1, r2, cpar, pre1, pre2, drain):
        wait_gather(p, cpar)
        if pre1:
            wait_idx(c + 1, r1)
            issue_gather(r1, 1 - cpar)
        compute(p, cpar)
        if drain:
            wait_scat(r2)
        if pre2:
            issue_idx(c + 2, r2)
        issue_scat(p)

    issue_idx(0, 0)
    wait_idx(0, 0)
    issue_gather(0, 0)
    issue_idx(1, 1)

    npeel = 6
    nmain = npeel + ((nchunks - npeel - 2) // 6) * 6
    for c in range(npeel):
        body_chunk(c, c % 3, (c + 1) % 3, (c + 2) % 3, c % 2,
                   True, True, c >= 1)

    def main(k, _):
        for j in range(6):
            jj = npeel + j
            body_chunk(npeel + k * 6 + j, jj % 3, (jj + 1) % 3, (jj + 2) % 3,
                       jj % 2, True, True, True)
        return 0

    lax.fori_loop(0, (nmain - npeel) // 6, main, 0)
    for c in range(nmain, nchunks):
        body_chunk(c, c % 3, (c + 1) % 3, (c + 2) % 3, c % 2,
                   c + 1 < nchunks, c + 2 < nchunks, True)
    wait_scat((nchunks - 1) % 3)
    plsc.subcore_barrier()

    r0 = s_ax * rows_per_s
    pltpu.sync_copy(mq_sh.at[pl.ds(r0, rows_per_s)],
                    mq_hbm.at[c_ax, pl.ds(r0, rows_per_s)])


def _combine_kernel(x_ref, msg_ref, sum_ref, sq_ref, den_ref, cnt_ref,
                    w_ref, bias_ref, o_ref):
    msg = msg_ref[...] / (den_ref[...] + 1e-16)
    inv = 1.0 / jnp.maximum(cnt_ref[...], 1.0)
    mean = sum_ref[...] * inv
    var = sq_ref[...] * inv - mean * mean
    h = jnp.concatenate([x_ref[...], msg, var], axis=1)
    o_ref[...] = (jnp.dot(h, w_ref[...], preferred_element_type=jnp.float32)
                  + bias_ref[0:1, :])


def kernel(x, edge_index, W_self, b_self, W_neigh, b_neigh, W_var, b_var):
    n, d = x.shape
    e = edge_index.shape[1]
    assert d == 128
    np_pad = ((n + NS * 16 - 1) // (NS * 16)) * (NS * 16)
    b1 = 40
    b2 = 80
    e_per_w = e // (NC * NS)
    e_per_s = e // NS
    assert e_per_w % b1 == 0 and e_per_s % b2 == 0

    src = edge_index[0]
    dst = edge_index[1]
    mesh = plsc.VectorSubcoreMesh(core_axis_name="c", subcore_axis_name="s")

    sc_params = pltpu.CompilerParams(needs_layout_passes=False,
                                     use_tc_tiling_on_sc=False)
    k1 = functools.partial(
        pl.kernel,
        compiler_params=sc_params,
        out_type=(
            jax.ShapeDtypeStruct((e,), jnp.float32),
            jax.ShapeDtypeStruct((NC, np_pad, 128), jnp.float32),
            jax.ShapeDtypeStruct((NC, np_pad), jnp.float32),
            jax.ShapeDtypeStruct((NC, np_pad), jnp.float32),
        ),
        mesh=mesh,
        scratch_types=(
            [pltpu.VMEM((b1, 128), jnp.float32)] * 5
            + [pltpu.VMEM((b1,), jnp.int32)] * 6
            + [pltpu.VMEM((b1,), jnp.float32)] * 3
            + [
                pltpu.VMEM((((b1 + L - 1) // L) * L,), jnp.float32),
                pltpu.VMEM((640,), jnp.float32),
                pltpu.VMEM_SHARED((np_pad, 128), jnp.float32),
                pltpu.VMEM_SHARED((np_pad,), jnp.float32),
                pltpu.VMEM_SHARED((np_pad,), jnp.float32),
            ]
            + [pltpu.SemaphoreType.DMA] * 12
        ),
    )(functools.partial(_edge_kernel, np_pad, e_per_w, e_per_w // b1, b1))
    eatt, sum_p, den_p, cnt_p = k1(x, src, dst)

    xcat = jnp.concatenate([x[:, :64], x[:, 64:]], axis=0)
    k2 = functools.partial(
        pl.kernel,
        compiler_params=sc_params,
        out_type=jax.ShapeDtypeStruct((NC, np_pad, 128), jnp.float32),
        mesh=mesh,
        scratch_types=(
            [pltpu.VMEM((b2,), jnp.int32)] * 9
            + [pltpu.VMEM((b2,), jnp.float32)] * 3
            + [pltpu.VMEM((b2, 64), jnp.float32)] * 2
            + [pltpu.VMEM((b2, 128), jnp.float32)] * 3
            + [pltpu.VMEM_SHARED((np_pad, 128), jnp.float32)]
            + [pltpu.SemaphoreType.DMA] * 9
        ),
    )(functools.partial(_feat_kernel, n, np_pad, e_per_s, e_per_s // b2, b2))
    mq_p = k2(xcat, src, dst, eatt)

    sum_src = (sum_p[0] + sum_p[1])[:n]
    den = (den_p[0] + den_p[1])[:n]
    cnt = (cnt_p[0] + cnt_p[1])[:n]
    msg_raw = jnp.concatenate([mq_p[0, :, :64], mq_p[1, :, :64]], axis=1)[:n]
    sumsq = jnp.concatenate([mq_p[0, :, 64:], mq_p[1, :, 64:]], axis=1)[:n]
    den_b = jnp.broadcast_to(den[:, None], (n, d))
    cnt_b = jnp.broadcast_to(cnt[:, None], (n, d))
    wcat = jnp.concatenate([W_self.T, W_neigh.T, W_var.T], axis=0)
    bias = jnp.broadcast_to((b_self + b_neigh + b_var)[None, :], (8, d))

    bn = 1000
    grid = n // bn
    out = pl.pallas_call(
        _combine_kernel,
        out_shape=jax.ShapeDtypeStruct((n, d), jnp.float32),
        grid=(grid,),
        in_specs=[
            pl.BlockSpec((bn, d), lambda i: (i, 0)),
            pl.BlockSpec((bn, d), lambda i: (i, 0)),
            pl.BlockSpec((bn, d), lambda i: (i, 0)),
            pl.BlockSpec((bn, d), lambda i: (i, 0)),
            pl.BlockSpec((bn, d), lambda i: (i, 0)),
            pl.BlockSpec((bn, d), lambda i: (i, 0)),
            pl.BlockSpec((3 * d, d), lambda i: (0, 0)),
            pl.BlockSpec((8, d), lambda i: (0, 0)),
        ],
        out_specs=pl.BlockSpec((bn, d), lambda i: (i, 0)),
    )(x, msg_raw, sum_src, sumsq, den_b, cnt_b, wcat, bias)
    return out


# speedup vs baseline: 8.6670x; 1.0034x over previous
import functools

import jax
import jax.numpy as jnp
from jax import lax
from jax.experimental import pallas as pl
from jax.experimental.pallas import tpu as pltpu
from jax.experimental.pallas import tpu_sc as plsc

NC = 2
NS = 16
L = 16

_SCALE = float(128) ** (-0.5)


def _zero_fill(buf, n_vecs):
    zeros = jnp.zeros((L,), jnp.float32)

    def body(k, _):
        buf[pl.ds(k * L, L)] = zeros
        return 0

    lax.fori_loop(0, n_vecs, body, 0)


def _zero_fill2d(buf, nrows, rowlen):
    zeros = jnp.zeros((L,), jnp.float32)

    def body(r, _):
        for j in range(rowlen // L):
            buf[r, pl.ds(j * L, L)] = zeros
        return 0

    lax.fori_loop(0, nrows, body, 0)


def _edge_kernel(np_pad, e_per_w, nchunks, b,
                 x_hbm, src_hbm, dst_hbm,
                 eatt_hbm, sum_hbm, den_hbm, cnt_hbm,
                 xs0, xs1, xs2, xd0, xd1, sv0, sv1, sv2,
                 dv0, dv1, dv2, eb0, eb1, eb2,
                 ones_v, zflat, sum_sh, den_sh, cnt_sh,
                 gs0, gs1, gs2, is0, is1, is2,
                 sl0, sl1, sl2, si0, si1, si2):
    c_ax = lax.axis_index("c")
    s_ax = lax.axis_index("s")
    w = c_ax * NS + s_ax
    rows_per_s = np_pad // NS

    xs = (xs0, xs1, xs2)
    xd = (xd0, xd1)
    sv = (sv0, sv1, sv2)
    dv = (dv0, dv1, dv2)
    eb = (eb0, eb1, eb2)
    gs = (gs0, gs1, gs2)
    isem = (is0, is1, is2)
    slin = (sl0, sl1, sl2)
    sind = (si0, si1, si2)

    _zero_fill2d(xs0, b, 128)
    _zero_fill(zflat, rows_per_s // L)
    for k in range((b + L - 1) // L):
        ones_v[pl.ds(k * L, L)] = jnp.ones((L,), jnp.float32)
    for t in range(rows_per_s // b):
        pltpu.sync_copy(xs0, sum_sh.at[pl.ds(s_ax * rows_per_s + t * b, b)])
    pltpu.sync_copy(zflat, den_sh.at[pl.ds(s_ax * rows_per_s, rows_per_s)])
    pltpu.sync_copy(zflat, cnt_sh.at[pl.ds(s_ax * rows_per_s, rows_per_s)])
    plsc.subcore_barrier()

    m15 = lax.broadcasted_iota(jnp.int32, (L,), 0) == 15

    def base_of(c):
        return w * e_per_w + c * b

    def issue_idx(c, r):
        pltpu.async_copy(src_hbm.at[pl.ds(base_of(c), b)], sv[r], isem[r])
        pltpu.async_copy(dst_hbm.at[pl.ds(base_of(c), b)], dv[r], isem[r])

    def wait_idx(c, r):
        pltpu.make_async_copy(src_hbm.at[pl.ds(base_of(c), b)], sv[r],
                              isem[r]).wait()
        pltpu.make_async_copy(dst_hbm.at[pl.ds(base_of(c), b)], dv[r],
                              isem[r]).wait()

    def issue_gather(r, par):
        pltpu.async_copy(x_hbm.at[sv[r]], xs[r], gs[r])
        pltpu.async_copy(x_hbm.at[dv[r]], xd[par], gs[r])

    def wait_gather(r, par):
        pltpu.make_async_copy(x_hbm.at[sv[r]], xs[r], gs[r]).wait()
        pltpu.make_async_copy(x_hbm.at[dv[r]], xd[par], gs[r]).wait()

    def issue_scat(c, p):
        pltpu.async_copy(eb[p], eatt_hbm.at[pl.ds(base_of(c), b)], slin[p])
        pltpu.async_copy(eb[p], den_sh.at[dv[p]], sind[p], add=True)
        pltpu.async_copy(ones_v.at[pl.ds(0, b)], cnt_sh.at[dv[p]], sind[p],
                         add=True)
        pltpu.async_copy(xs[p], sum_sh.at[dv[p]], sind[p], add=True)

    def wait_scat(c, p):
        pltpu.make_async_copy(eb[p], eatt_hbm.at[pl.ds(base_of(c), b)],
                              slin[p]).wait()
        pltpu.make_async_copy(eb[p], den_sh.at[dv[p]], sind[p]).wait()
        pltpu.make_async_copy(ones_v.at[pl.ds(0, b)], cnt_sh.at[dv[p]],
                              sind[p]).wait()
        pltpu.make_async_copy(xs[p], sum_sh.at[dv[p]], sind[p]).wait()

    def compute(p, par):
        xsp, xdp, ebp = xs[p], xd[par], eb[p]

        def edge(e, _):
            acc = xsp[e, pl.ds(0, L)] * xdp[e, pl.ds(0, L)]
            for j in range(1, 8):
                acc = acc + xsp[e, pl.ds(j * L, L)] * xdp[e, pl.ds(j * L, L)]
            sc = plsc.cumsum(acc)
            ev = jnp.exp(sc * _SCALE)
            plsc.store_scatter(ebp, [jnp.full((L,), e, jnp.int32)], ev,
                               mask=m15)
            return 0

        lax.fori_loop(0, b, edge, 0, unroll=4)

    def body_chunk(c, p, r1, r2, cpar, pre1, pre2, drain):
        wait_gather(p, cpar)
        if pre1:
            wait_idx(c + 1, r1)
            issue_gather(r1, 1 - cpar)
        compute(p, cpar)
        if drain:
            wait_scat(c - 1, r2)
        if pre2:
            issue_idx(c + 2, r2)
        issue_scat(c, p)

    issue_idx(0, 0)
    wait_idx(0, 0)
    issue_gather(0, 0)
    issue_idx(1, 1)

    npeel = 6
    nmain = npeel + ((nchunks - npeel - 2) // 6) * 6
    for c in range(npeel):
        body_chunk(c, c % 3, (c + 1) % 3, (c + 2) % 3, c % 2,
                   True, True, c >= 1)

    def main(k, _):
        for j in range(6):
            c = npeel + k * 6 + j
            jj = npeel + j
            body_chunk(c, jj % 3, (jj + 1) % 3, (jj + 2) % 3, jj % 2,
                       True, True, True)
        return 0

    lax.fori_loop(0, (nmain - npeel) // 6, main, 0)
    for c in range(nmain, nchunks):
        body_chunk(c, c % 3, (c + 1) % 3, (c + 2) % 3, c % 2,
                   c + 1 < nchunks, c + 2 < nchunks, True)
    wait_scat(nchunks - 1, (nchunks - 1) % 3)
    plsc.subcore_barrier()

    r0 = s_ax * rows_per_s
    pltpu.sync_copy(sum_sh.at[pl.ds(r0, rows_per_s)],
                    sum_hbm.at[c_ax, pl.ds(r0, rows_per_s)])
    pltpu.sync_copy(den_sh.at[pl.ds(r0, rows_per_s)],
                    den_hbm.at[c_ax, pl.ds(r0, rows_per_s)])
    pltpu.sync_copy(cnt_sh.at[pl.ds(r0, rows_per_s)],
                    cnt_hbm.at[c_ax, pl.ds(r0, rows_per_s)])


def _feat_kernel(n, np_pad, e_per_s, nchunks, b,
                 xcat_hbm, src_hbm, dst_hbm, eatt_hbm,
                 mq_hbm,
                 sv0, sv1, sv2, gv0, gv1, gv2, dv0, dv1, dv2,
                 eb0, eb1, eb2, xs0, xs1, st0, st1, st2,
                 mq_sh,
                 gs0, gs1, gs2, is0, is1, is2, si0, si1, si2):
    c_ax = lax.axis_index("c")
    s_ax = lax.axis_index("s")
    rows_per_s = np_pad // NS

    sv = (sv0, sv1, sv2)
    gv = (gv0, gv1, gv2)
    dv = (dv0, dv1, dv2)
    eb = (eb0, eb1, eb2)
    xs = (xs0, xs1)
    st = (st0, st1, st2)
    gs = (gs0, gs1, gs2)
    isem = (is0, is1, is2)
    sind = (si0, si1, si2)

    _zero_fill2d(st0, b, 128)
    for t in range(rows_per_s // b):
        pltpu.sync_copy(st0, mq_sh.at[pl.ds(s_ax * rows_per_s + t * b, b)])
    plsc.subcore_barrier()

    off = c_ax * n

    def base_of(c):
        return s_ax * e_per_s + c * b

    def issue_idx(c, r):
        pltpu.async_copy(src_hbm.at[pl.ds(base_of(c), b)], sv[r], isem[r])
        pltpu.async_copy(dst_hbm.at[pl.ds(base_of(c), b)], dv[r], isem[r])
        pltpu.async_copy(eatt_hbm.at[pl.ds(base_of(c), b)], eb[r], isem[r])

    def wait_idx(c, r):
        pltpu.make_async_copy(src_hbm.at[pl.ds(base_of(c), b)], sv[r],
                              isem[r]).wait()
        pltpu.make_async_copy(dst_hbm.at[pl.ds(base_of(c), b)], dv[r],
                              isem[r]).wait()
        pltpu.make_async_copy(eatt_hbm.at[pl.ds(base_of(c), b)], eb[r],
                              isem[r]).wait()

    def issue_gather(r, par):
        for k in range(b // L):
            gv[r][pl.ds(k * L, L)] = sv[r][pl.ds(k * L, L)] + off
        pltpu.async_copy(xcat_hbm.at[gv[r]], xs[par], gs[r])

    def wait_gather(r, par):
        pltpu.make_async_copy(xcat_hbm.at[gv[r]], xs[par], gs[r]).wait()

    def issue_scat(p):
        pltpu.async_copy(st[p], mq_sh.at[dv[p]], sind[p], add=True)

    def wait_scat(p):
        pltpu.make_async_copy(st[p], mq_sh.at[dv[p]], sind[p]).wait()

    def compute(p, par):
        xsp, ebp, stp = xs[par], eb[p], st[p]

        def edge(e, _):
            ebc = plsc.load_gather(ebp, [jnp.full((L,), e, jnp.int32)])
            for j in range(4):
                xsj = xsp[e, pl.ds(j * L, L)]
                stp[e, pl.ds(j * L, L)] = ebc * xsj
                stp[e, pl.ds(64 + j * L, L)] = xsj * xsj
            return 0

        lax.fori_loop(0, b, edge, 0, unroll=4)

    def body_chunk(c, p, r1, r2, cpar, pre1, pre2, drain):
        wait_gather(p, cpar)
        if pre1:
            wait_idx(c + 1, r1)
            issue_gather(r1, 1 - cpar)
        compute(p, cpar)
        if drain:
            wait_scat(r2)
        if pre2:
            issue_idx(c + 2, r2)
        issue_scat(p)

    issue_idx(0, 0)
    wait_idx(0, 0)
    issue_gather(0, 0)
    issue_idx(1, 1)

    npeel = 6
    nmain = npeel + ((nchunks - npeel - 2) // 6) * 6
    for c in range(npeel):
        body_chunk(c, c % 3, (c + 1) % 3, (c + 2) % 3, c % 2,
                   True, True, c >= 1)

    def main(k, _):
        for j in range(6):
            jj = npeel + j
            body_chunk(npeel + k * 6 + j, jj % 3, (jj + 1) % 3, (jj + 2) % 3,
                       jj % 2, True, True, True)
        return 0

    lax.fori_loop(0, (nmain - npeel) // 6, main, 0)
    for c in range(nmain, nchunks):
        body_chunk(c, c % 3, (c + 1) % 3, (c + 2) % 3, c % 2,
                   c + 1 < nchunks, c + 2 < nchunks, True)
    wait_scat((nchunks - 1) % 3)
    plsc.subcore_barrier()

    r0 = s_ax * rows_per_s
    pltpu.sync_copy(mq_sh.at[pl.ds(r0, rows_per_s)],
                    mq_hbm.at[c_ax, pl.ds(r0, rows_per_s)])


def _combine_kernel(x_ref, msg_ref, sum_ref, sq_ref, den_ref, cnt_ref,
                    w_ref, bias_ref, o_ref):
    msg = msg_ref[...] / (den_ref[...] + 1e-16)
    inv = 1.0 / jnp.maximum(cnt_ref[...], 1.0)
    mean = sum_ref[...] * inv
    var = sq_ref[...] * inv - mean * mean
    h = jnp.concatenate([x_ref[...], msg, var], axis=1)
    o_ref[...] = (jnp.dot(h, w_ref[...], preferred_element_type=jnp.float32)
                  + bias_ref[0:1, :])


def kernel(x, edge_index, W_self, b_self, W_neigh, b_neigh, W_var, b_var):
    n, d = x.shape
    e = edge_index.shape[1]
    assert d == 128
    np_pad = ((n + NS * 16 - 1) // (NS * 16)) * (NS * 16)
    b1 = 40
    b2 = 80
    e_per_w = e // (NC * NS)
    e_per_s = e // NS
    assert e_per_w % b1 == 0 and e_per_s % b2 == 0

    src = edge_index[0]
    dst = edge_index[1]
    mesh = plsc.VectorSubcoreMesh(core_axis_name="c", subcore_axis_name="s")

    sc_params = pltpu.CompilerParams(needs_layout_passes=False,
                                     use_tc_tiling_on_sc=False)
    k1 = functools.partial(
        pl.kernel,
        compiler_params=sc_params,
        out_type=(
            jax.ShapeDtypeStruct((e,), jnp.float32),
            jax.ShapeDtypeStruct((NC, np_pad, 128), jnp.float32),
            jax.ShapeDtypeStruct((NC, np_pad), jnp.float32),
            jax.ShapeDtypeStruct((NC, np_pad), jnp.float32),
        ),
        mesh=mesh,
        scratch_types=(
            [pltpu.VMEM((b1, 128), jnp.float32)] * 5
            + [pltpu.VMEM((b1,), jnp.int32)] * 6
            + [pltpu.VMEM((b1,), jnp.float32)] * 3
            + [
                pltpu.VMEM((((b1 + L - 1) // L) * L,), jnp.float32),
                pltpu.VMEM((640,), jnp.float32),
                pltpu.VMEM_SHARED((np_pad, 128), jnp.float32),
                pltpu.VMEM_SHARED((np_pad,), jnp.float32),
                pltpu.VMEM_SHARED((np_pad,), jnp.float32),
            ]
            + [pltpu.SemaphoreType.DMA] * 12
        ),
    )(functools.partial(_edge_kernel, np_pad, e_per_w, e_per_w // b1, b1))
    eatt, sum_p, den_p, cnt_p = k1(x, src, dst)

    xcat = jnp.concatenate([x[:, :64], x[:, 64:]], axis=0)
    k2 = functools.partial(
        pl.kernel,
        compiler_params=sc_params,
        out_type=jax.ShapeDtypeStruct((NC, np_pad, 128), jnp.float32),
        mesh=mesh,
        scratch_types=(
            [pltpu.VMEM((b2,), jnp.int32)] * 9
            + [pltpu.VMEM((b2,), jnp.float32)] * 3
            + [pltpu.VMEM((b2, 64), jnp.float32)] * 2
            + [pltpu.VMEM((b2, 128), jnp.float32)] * 3
            + [pltpu.VMEM_SHARED((np_pad, 128), jnp.float32)]
            + [pltpu.SemaphoreType.DMA] * 9
        ),
    )(functools.partial(_feat_kernel, n, np_pad, e_per_s, e_per_s // b2, b2))
    mq_p = k2(xcat, src, dst, eatt)

    sum_src = (sum_p[0] + sum_p[1])[:n]
    den = (den_p[0] + den_p[1])[:n]
    cnt = (cnt_p[0] + cnt_p[1])[:n]
    msg_raw = jnp.concatenate([mq_p[0, :, :64], mq_p[1, :, :64]], axis=1)[:n]
    sumsq = jnp.concatenate([mq_p[0, :, 64:], mq_p[1, :, 64:]], axis=1)[:n]
    den_b = jnp.broadcast_to(den[:, None], (n, d))
    cnt_b = jnp.broadcast_to(cnt[:, None], (n, d))
    wcat = jnp.concatenate([W_self.T, W_neigh.T, W_var.T], axis=0)
    bias = jnp.broadcast_to((b_self + b_neigh + b_var)[None, :], (8, d))

    bn = 1000
    grid = n // bn
    out = pl.pallas_call(
        _combine_kernel,
        out_shape=jax.ShapeDtypeStruct((n, d), jnp.float32),
        grid=(grid,),
        in_specs=[
            pl.BlockSpec((bn, d), lambda i: (i, 0)),
            pl.BlockSpec((bn, d), lambda i: (i, 0)),
            pl.BlockSpec((bn, d), lambda i: (i, 0)),
            pl.BlockSpec((bn, d), lambda i: (i, 0)),
            pl.BlockSpec((bn, d), lambda i: (i, 0)),
            pl.BlockSpec((bn, d), lambda i: (i, 0)),
            pl.BlockSpec((3 * d, d), lambda i: (0, 0)),
            pl.BlockSpec((8, d), lambda i: (0, 0)),
        ],
        out_specs=pl.BlockSpec((bn, d), lambda i: (i, 0)),
    )(x, msg_raw, sum_src, sumsq, den_b, cnt_b, wcat, bias)
    return out


# speedup vs baseline: 8.6924x; 1.0029x over previous
import functools

import jax
import jax.numpy as jnp
from jax import lax
from jax.experimental import pallas as pl
from jax.experimental.pallas import tpu as pltpu
from jax.experimental.pallas import tpu_sc as plsc

NC = 2
NS = 16
L = 16

_SCALE = float(128) ** (-0.5)


def _zero_fill(buf, n_vecs):
    zeros = jnp.zeros((L,), jnp.float32)

    def body(k, _):
        buf[pl.ds(k * L, L)] = zeros
        return 0

    lax.fori_loop(0, n_vecs, body, 0)


def _zero_fill2d(buf, nrows, rowlen):
    zeros = jnp.zeros((L,), jnp.float32)

    def body(r, _):
        for j in range(rowlen // L):
            buf[r, pl.ds(j * L, L)] = zeros
        return 0

    lax.fori_loop(0, nrows, body, 0)


def _edge_kernel(np_pad, e_per_w, nchunks, b,
                 x_hbm, src_hbm, dst_hbm,
                 eatt_hbm, sum_hbm, den_hbm, cnt_hbm,
                 xs0, xs1, xs2, xs3, xd0, xd1, xd2, xd3,
                 sv0, sv1, sv2, sv3, dv0, dv1, dv2, dv3,
                 eb0, eb1, eb2, eb3,
                 ones_v, zflat, sum_sh, den_sh, cnt_sh,
                 gs0, gs1, gs2, gs3, is0, is1, is2, is3,
                 sl0, sl1, sl2, sl3, si0, si1, si2, si3):
    c_ax = lax.axis_index("c")
    s_ax = lax.axis_index("s")
    w = c_ax * NS + s_ax
    rows_per_s = np_pad // NS

    xs = (xs0, xs1, xs2, xs3)
    xd = (xd0, xd1, xd2, xd3)
    sv = (sv0, sv1, sv2, sv3)
    dv = (dv0, dv1, dv2, dv3)
    eb = (eb0, eb1, eb2, eb3)
    gs = (gs0, gs1, gs2, gs3)
    isem = (is0, is1, is2, is3)
    slin = (sl0, sl1, sl2, sl3)
    sind = (si0, si1, si2, si3)

    _zero_fill2d(xs0, b, 128)
    _zero_fill(zflat, rows_per_s // L)
    for k in range((b + L - 1) // L):
        ones_v[pl.ds(k * L, L)] = jnp.ones((L,), jnp.float32)
    for t in range(rows_per_s // b):
        pltpu.sync_copy(xs0, sum_sh.at[pl.ds(s_ax * rows_per_s + t * b, b)])
    pltpu.sync_copy(zflat, den_sh.at[pl.ds(s_ax * rows_per_s, rows_per_s)])
    pltpu.sync_copy(zflat, cnt_sh.at[pl.ds(s_ax * rows_per_s, rows_per_s)])
    plsc.subcore_barrier()

    m15 = lax.broadcasted_iota(jnp.int32, (L,), 0) == 15

    def base_of(c):
        return w * e_per_w + c * b

    def issue_idx(c, r):
        pltpu.async_copy(src_hbm.at[pl.ds(base_of(c), b)], sv[r], isem[r])
        pltpu.async_copy(dst_hbm.at[pl.ds(base_of(c), b)], dv[r], isem[r])

    def wait_idx(c, r):
        pltpu.make_async_copy(src_hbm.at[pl.ds(base_of(c), b)], sv[r],
                              isem[r]).wait()
        pltpu.make_async_copy(dst_hbm.at[pl.ds(base_of(c), b)], dv[r],
                              isem[r]).wait()

    def issue_gather(r):
        pltpu.async_copy(x_hbm.at[sv[r]], xs[r], gs[r])
        pltpu.async_copy(x_hbm.at[dv[r]], xd[r], gs[r])

    def wait_gather(r):
        pltpu.make_async_copy(x_hbm.at[sv[r]], xs[r], gs[r]).wait()
        pltpu.make_async_copy(x_hbm.at[dv[r]], xd[r], gs[r]).wait()

    def issue_scat(c, p):
        pltpu.async_copy(eb[p], eatt_hbm.at[pl.ds(base_of(c), b)], slin[p])
        pltpu.async_copy(eb[p], den_sh.at[dv[p]], sind[p], add=True)
        pltpu.async_copy(ones_v.at[pl.ds(0, b)], cnt_sh.at[dv[p]], sind[p],
                         add=True)
        pltpu.async_copy(xs[p], sum_sh.at[dv[p]], sind[p], add=True)

    def wait_scat(c, p):
        pltpu.make_async_copy(eb[p], eatt_hbm.at[pl.ds(base_of(c), b)],
                              slin[p]).wait()
        pltpu.make_async_copy(eb[p], den_sh.at[dv[p]], sind[p]).wait()
        pltpu.make_async_copy(ones_v.at[pl.ds(0, b)], cnt_sh.at[dv[p]],
                              sind[p]).wait()
        pltpu.make_async_copy(xs[p], sum_sh.at[dv[p]], sind[p]).wait()

    def compute(p):
        xsp, xdp, ebp = xs[p], xd[p], eb[p]

        def edge(e, _):
            acc = xsp[e, pl.ds(0, L)] * xdp[e, pl.ds(0, L)]
            for j in range(1, 8):
                acc = acc + xsp[e, pl.ds(j * L, L)] * xdp[e, pl.ds(j * L, L)]
            sc = plsc.cumsum(acc)
            ev = jnp.exp(sc * _SCALE)
            plsc.store_scatter(ebp, [jnp.full((L,), e, jnp.int32)], ev,
                               mask=m15)
            return 0

        lax.fori_loop(0, b, edge, 0, unroll=4)

    def body_chunk(c, p, r2, r3, pre2, pre3, drain):
        wait_gather(p)
        if pre2:
            wait_idx(c + 2, r2)
            issue_gather(r2)
        compute(p)
        if drain:
            wait_scat(c - 1, r3)
        if pre3:
            issue_idx(c + 3, r3)
        issue_scat(c, p)

    issue_idx(0, 0)
    issue_idx(1, 1)
    wait_idx(0, 0)
    issue_gather(0)
    wait_idx(1, 1)
    issue_gather(1)
    issue_idx(2, 2)

    npeel = 4
    nmain = npeel + ((nchunks - npeel - 3) // 4) * 4
    for c in range(npeel):
        body_chunk(c, c % 4, (c + 2) % 4, (c + 3) % 4, True, True, c >= 1)

    def main(k, _):
        for j in range(4):
            c = npeel + k * 4 + j
            body_chunk(c, j % 4, (j + 2) % 4, (j + 3) % 4, True, True, True)
        return 0

    lax.fori_loop(0, (nmain - npeel) // 4, main, 0)
    for c in range(nmain, nchunks):
        body_chunk(c, c % 4, (c + 2) % 4, (c + 3) % 4,
                   c + 2 < nchunks, c + 3 < nchunks, True)
    wait_scat(nchunks - 1, (nchunks - 1) % 4)
    plsc.subcore_barrier()

    r0 = s_ax * rows_per_s
    pltpu.sync_copy(sum_sh.at[pl.ds(r0, rows_per_s)],
                    sum_hbm.at[c_ax, pl.ds(r0, rows_per_s)])
    pltpu.sync_copy(den_sh.at[pl.ds(r0, rows_per_s)],
                    den_hbm.at[c_ax, pl.ds(r0, rows_per_s)])
    pltpu.sync_copy(cnt_sh.at[pl.ds(r0, rows_per_s)],
                    cnt_hbm.at[c_ax, pl.ds(r0, rows_per_s)])


def _feat_kernel(n, np_pad, e_per_s, nchunks, b,
                 xcat_hbm, src_hbm, dst_hbm, eatt_hbm,
                 mq_hbm,
                 sv0, sv1, sv2, sv3, gv0, gv1, gv2, gv3,
                 dv0, dv1, dv2, dv3, eb0, eb1, eb2, eb3,
                 xs0, xs1, xs2, st0, st1, st2,
                 mq_sh,
                 gs0, gs1, gs2, gs3, is0, is1, is2, is3, si0, si1, si2):
    c_ax = lax.axis_index("c")
    s_ax = lax.axis_index("s")
    rows_per_s = np_pad // NS

    sv = (sv0, sv1, sv2, sv3)
    gv = (gv0, gv1, gv2, gv3)
    dv = (dv0, dv1, dv2, dv3)
    eb = (eb0, eb1, eb2, eb3)
    xs = (xs0, xs1, xs2)
    st = (st0, st1, st2)
    gs = (gs0, gs1, gs2, gs3)
    isem = (is0, is1, is2, is3)
    sind = (si0, si1, si2)

    _zero_fill2d(st0, b, 128)
    for t in range(rows_per_s // b):
        pltpu.sync_copy(st0, mq_sh.at[pl.ds(s_ax * rows_per_s + t * b, b)])
    plsc.subcore_barrier()

    off = c_ax * n

    def base_of(c):
        return s_ax * e_per_s + c * b

    def issue_idx(c, r):
        pltpu.async_copy(src_hbm.at[pl.ds(base_of(c), b)], sv[r], isem[r])
        pltpu.async_copy(dst_hbm.at[pl.ds(base_of(c), b)], dv[r], isem[r])
        pltpu.async_copy(eatt_hbm.at[pl.ds(base_of(c), b)], eb[r], isem[r])

    def wait_idx(c, r):
        pltpu.make_async_copy(src_hbm.at[pl.ds(base_of(c), b)], sv[r],
                              isem[r]).wait()
        pltpu.make_async_copy(dst_hbm.at[pl.ds(base_of(c), b)], dv[r],
                              isem[r]).wait()
        pltpu.make_async_copy(eatt_hbm.at[pl.ds(base_of(c), b)], eb[r],
                              isem[r]).wait()

    def issue_gather(r4, r3):
        for k in range(b // L):
            gv[r4][pl.ds(k * L, L)] = sv[r4][pl.ds(k * L, L)] + off
        pltpu.async_copy(xcat_hbm.at[gv[r4]], xs[r3], gs[r4])

    def wait_gather(r4, r3):
        pltpu.make_async_copy(xcat_hbm.at[gv[r4]], xs[r3], gs[r4]).wait()

    def issue_scat(p3, p4):
        pltpu.async_copy(st[p3], mq_sh.at[dv[p4]], sind[p3], add=True)

    def wait_scat(p3, p4):
        pltpu.make_async_copy(st[p3], mq_sh.at[dv[p4]], sind[p3]).wait()

    def compute(p3, p4):
        xsp, ebp, stp = xs[p3], eb[p4], st[p3]

        def edge(e, _):
            ebc = plsc.load_gather(ebp, [jnp.full((L,), e, jnp.int32)])
            for j in range(4):
                xsj = xsp[e, pl.ds(j * L, L)]
                stp[e, pl.ds(j * L, L)] = ebc * xsj
                stp[e, pl.ds(64 + j * L, L)] = xsj * xsj
            return 0

        lax.fori_loop(0, b, edge, 0, unroll=4)

    def body_chunk(c, p3, p4, pre2, pre3, drain):
        wait_gather(p4, p3)
        if pre2:
            wait_idx(c + 2, (p4 + 2) % 4)
            issue_gather((p4 + 2) % 4, (p3 + 2) % 3)
        compute(p3, p4)
        if drain:
            wait_scat((p3 + 2) % 3, (p4 + 3) % 4)
        if pre3:
            issue_idx(c + 3, (p4 + 3) % 4)
        issue_scat(p3, p4)

    issue_idx(0, 0)
    issue_idx(1, 1)
    wait_idx(0, 0)
    issue_gather(0, 0)
    wait_idx(1, 1)
    issue_gather(1, 1)
    issue_idx(2, 2)

    npeel = 4
    nmain = npeel + ((nchunks - npeel - 3) // 12) * 12
    for c in range(npeel):
        body_chunk(c, c % 3, c % 4, True, True, c >= 1)

    def main(k, _):
        for j in range(12):
            c = npeel + k * 12 + j
            body_chunk(c, (npeel + j) % 3, j % 4, True, True, True)
        return 0

    lax.fori_loop(0, (nmain - npeel) // 12, main, 0)
    for c in range(nmain, nchunks):
        body_chunk(c, c % 3, c % 4, c + 2 < nchunks, c + 3 < nchunks, True)
    wait_scat((nchunks - 1) % 3, (nchunks - 1) % 4)
    plsc.subcore_barrier()

    r0 = s_ax * rows_per_s
    pltpu.sync_copy(mq_sh.at[pl.ds(r0, rows_per_s)],
                    mq_hbm.at[c_ax, pl.ds(r0, rows_per_s)])


def _combine_kernel(x_ref, msg_ref, sum_ref, sq_ref, den_ref, cnt_ref,
                    w_ref, bias_ref, o_ref):
    msg = msg_ref[...] / (den_ref[...] + 1e-16)
    inv = 1.0 / jnp.maximum(cnt_ref[...], 1.0)
    mean = sum_ref[...] * inv
    var = sq_ref[...] * inv - mean * mean
    h = jnp.concatenate([x_ref[...], msg, var], axis=1)
    o_ref[...] = (jnp.dot(h, w_ref[...], preferred_element_type=jnp.float32)
                  + bias_ref[0:1, :])


def kernel(x, edge_index, W_self, b_self, W_neigh, b_neigh, W_var, b_var):
    n, d = x.shape
    e = edge_index.shape[1]
    assert d == 128
    np_pad = ((n + NS * 16 - 1) // (NS * 16)) * (NS * 16)
    b1 = 40
    b2 = 80
    e_per_w = e // (NC * NS)
    e_per_s = e // NS
    assert e_per_w % b1 == 0 and e_per_s % b2 == 0

    src = edge_index[0]
    dst = edge_index[1]
    mesh = plsc.VectorSubcoreMesh(core_axis_name="c", subcore_axis_name="s")

    sc_params = pltpu.CompilerParams(needs_layout_passes=False,
                                     use_tc_tiling_on_sc=False)
    k1 = functools.partial(
        pl.kernel,
        compiler_params=sc_params,
        out_type=(
            jax.ShapeDtypeStruct((e,), jnp.float32),
            jax.ShapeDtypeStruct((NC, np_pad, 128), jnp.float32),
            jax.ShapeDtypeStruct((NC, np_pad), jnp.float32),
            jax.ShapeDtypeStruct((NC, np_pad), jnp.float32),
        ),
        mesh=mesh,
        scratch_types=(
            [pltpu.VMEM((b1, 128), jnp.float32)] * 8
            + [pltpu.VMEM((b1,), jnp.int32)] * 8
            + [pltpu.VMEM((b1,), jnp.float32)] * 4
            + [
                pltpu.VMEM((((b1 + L - 1) // L) * L,), jnp.float32),
                pltpu.VMEM((640,), jnp.float32),
                pltpu.VMEM_SHARED((np_pad, 128), jnp.float32),
                pltpu.VMEM_SHARED((np_pad,), jnp.float32),
                pltpu.VMEM_SHARED((np_pad,), jnp.float32),
            ]
            + [pltpu.SemaphoreType.DMA] * 16
        ),
    )(functools.partial(_edge_kernel, np_pad, e_per_w, e_per_w // b1, b1))
    eatt, sum_p, den_p, cnt_p = k1(x, src, dst)

    xcat = jnp.concatenate([x[:, :64], x[:, 64:]], axis=0)
    k2 = functools.partial(
        pl.kernel,
        compiler_params=sc_params,
        out_type=jax.ShapeDtypeStruct((NC, np_pad, 128), jnp.float32),
        mesh=mesh,
        scratch_types=(
            [pltpu.VMEM((b2,), jnp.int32)] * 12
            + [pltpu.VMEM((b2,), jnp.float32)] * 4
            + [pltpu.VMEM((b2, 64), jnp.float32)] * 3
            + [pltpu.VMEM((b2, 128), jnp.float32)] * 3
            + [pltpu.VMEM_SHARED((np_pad, 128), jnp.float32)]
            + [pltpu.SemaphoreType.DMA] * 11
        ),
    )(functools.partial(_feat_kernel, n, np_pad, e_per_s, e_per_s // b2, b2))
    mq_p = k2(xcat, src, dst, eatt)

    sum_src = (sum_p[0] + sum_p[1])[:n]
    den = (den_p[0] + den_p[1])[:n]
    cnt = (cnt_p[0] + cnt_p[1])[:n]
    msg_raw = jnp.concatenate([mq_p[0, :, :64], mq_p[1, :, :64]], axis=1)[:n]
    sumsq = jnp.concatenate([mq_p[0, :, 64:], mq_p[1, :, 64:]], axis=1)[:n]
    den_b = jnp.broadcast_to(den[:, None], (n, d))
    cnt_b = jnp.broadcast_to(cnt[:, None], (n, d))
    wcat = jnp.concatenate([W_self.T, W_neigh.T, W_var.T], axis=0)
    bias = jnp.broadcast_to((b_self + b_neigh + b_var)[None, :], (8, d))

    bn = 1000
    grid = n // bn
    out = pl.pallas_call(
        _combine_kernel,
        out_shape=jax.ShapeDtypeStruct((n, d), jnp.float32),
        grid=(grid,),
        in_specs=[
            pl.BlockSpec((bn, d), lambda i: (i, 0)),
            pl.BlockSpec((bn, d), lambda i: (i, 0)),
            pl.BlockSpec((bn, d), lambda i: (i, 0)),
            pl.BlockSpec((bn, d), lambda i: (i, 0)),
            pl.BlockSpec((bn, d), lambda i: (i, 0)),
            pl.BlockSpec((bn, d), lambda i: (i, 0)),
            pl.BlockSpec((3 * d, d), lambda i: (0, 0)),
            pl.BlockSpec((8, d), lambda i: (0, 0)),
        ],
        out_specs=pl.BlockSpec((bn, d), lambda i: (i, 0)),
    )(x, msg_raw, sum_src, sumsq, den_b, cnt_b, wcat, bias)
    return out
